# baseline (speedup 1.0000x reference)
import functools
import math

import jax
import jax.numpy as jnp
from jax import lax
from jax.experimental import pallas as pl
from jax.experimental.pallas import tpu as pltpu
from jax.experimental.pallas import tpu_sc as plsc

_D = 64
_SCALE = math.sqrt(_D)
_NC = 2
_NS = 16
_NW = _NC * _NS
_L = 16
_B = 4096
_T = 200
_TT = 4
_TR = _T // 8
_NCHUNK = _T // _TT


def _sc_body(idx_hbm, base_hbm, tab_hbm, out_hbm,
             idx_v, xbuf, ebuf, obuf, sem_x, sem_e):
    wid = lax.axis_index("s") * _NC + lax.axis_index("c")
    lane_iota = lax.iota(jnp.int32, _L)

    @pl.loop(0, _TR)
    def _stage(tr):
        pltpu.sync_copy(idx_hbm.at[tr, wid], idx_v.at[tr])

    @pl.loop(0, _NCHUNK)
    def _chunk_loop(g):
        tr = g // 2
        half = g % 2
        idx_slice = idx_v.at[tr, pl.ds(half * (_TT * 128), _TT * 128)]
        cp_x = pltpu.async_copy(base_hbm.at[idx_slice], xbuf, sem_x)
        cp_e = pltpu.async_copy(tab_hbm.at[idx_slice], ebuf, sem_e)
        cp_x.wait()
        cp_e.wait()

        @pl.loop(0, _TT * 8)
        def _tile_loop(i):
            s_loc = i // 8
            tc = i % 8
            row0 = s_loc * 128
            for cs in range(8):
                col = jnp.full((_L,), tc * 8 + cs, jnp.int32)
                for j in range(8):
                    rows = row0 + j * _L + lane_iota
                    xv = plsc.load_gather(xbuf, [rows, col])
                    ev = plsc.load_gather(ebuf, [rows, col])
                    obuf[i, pl.ds(cs * 128 + j * _L, _L)] = xv + _SCALE * ev

        @pl.loop(0, _TT * 8)
        def _write_loop(i):
            s_loc = i // 8
            tc = i % 8
            t = tr * 8 + half * _TT + s_loc
            pltpu.sync_copy(obuf.at[i], out_hbm.at[t, tc, wid])


def _make_sc_kernel():
    mesh = plsc.VectorSubcoreMesh(
        core_axis_name="c", subcore_axis_name="s",
        num_cores=_NC, num_subcores=_NS)
    return pl.kernel(
        _sc_body,
        out_type=jax.ShapeDtypeStruct((_T, 8, _NW, 1024), jnp.float32),
        mesh=mesh,
        compiler_params=pltpu.CompilerParams(
            use_tc_tiling_on_sc=False, needs_layout_passes=False),
        scratch_types=[
            pltpu.VMEM((_TR, 1024), jnp.int32),
            pltpu.VMEM((_TT * 128, _D), jnp.float32),
            pltpu.VMEM((_TT * 128, _D), jnp.float32),
            pltpu.VMEM((_TT * 8, 1024), jnp.float32),
            pltpu.SemaphoreType.DMA,
            pltpu.SemaphoreType.DMA,
        ],
    )


def kernel(tokens, base_table, table):
    t4 = (jnp.asarray(tokens, jnp.int32)
          .reshape(_NW, 128, _TR, 8)
          .transpose(2, 0, 3, 1)
          .reshape(_TR, _NW, 1024))
    out5 = _make_sc_kernel()(t4, base_table, table)
    return (out5.reshape(_T, 8, _NW, 8, 128)
            .transpose(2, 4, 0, 1, 3)
            .reshape(_B, _T, _D))


# speedup vs baseline: 1.0925x; 1.0925x over previous
import math

import jax
import jax.numpy as jnp
from jax import lax
from jax.experimental import pallas as pl
from jax.experimental.pallas import tpu as pltpu
from jax.experimental.pallas import tpu_sc as plsc

_D = 64
_SCALE = math.sqrt(_D)
_NC = 2
_NS = 16
_NW = _NC * _NS
_L = 16
_B = 4096
_T = 200
_TT = 2
_RC = _TT * 128
_TR = _T // 8
_NCHUNK = _T // _TT
_NTILE = _TT * 8


def _sc_body(idx_hbm, base_hbm, tab_hbm, out_hbm,
             idx_v, xbufs, ebufs, obufs, sems_x, sems_e, sems_w):
    wid = lax.axis_index("s") * _NC + lax.axis_index("c")
    lane_iota = lax.iota(jnp.int32, _L)

    def idx_slice(c):
        return idx_v.at[c // 4, pl.ds((c % 4) * _RC, _RC)]

    def fire_gathers(c, par):
        pltpu.async_copy(base_hbm.at[idx_slice(c)], xbufs[par], sems_x[par])
        pltpu.async_copy(tab_hbm.at[idx_slice(c)], ebufs[par], sems_e[par])

    def wait_gathers(c, par):
        pltpu.make_async_copy(base_hbm.at[idx_slice(c)], xbufs[par],
                              sems_x[par]).wait()
        pltpu.make_async_copy(tab_hbm.at[idx_slice(c)], ebufs[par],
                              sems_e[par]).wait()

    def drain_writes(par):
        for i in range(_NTILE):
            pltpu.make_async_copy(obufs[par].at[0], out_hbm.at[0, 0, wid],
                                  sems_w[par]).wait()

    def do_chunk(c, par):
        @pl.when(c + 1 < _NCHUNK)
        def _():
            fire_gathers(c + 1, 1 - par)

        wait_gathers(c, par)

        @pl.when(c >= 2)
        def _():
            drain_writes(par)

        xbuf, ebuf, obuf = xbufs[par], ebufs[par], obufs[par]

        @pl.loop(0, _NTILE)
        def _tile_loop(i):
            s_loc = i // 8
            tc = i % 8
            row0 = s_loc * 128
            rows = [row0 + j * _L + lane_iota for j in range(8)]
            for cs in range(8):
                col = jnp.full((_L,), tc * 8 + cs, jnp.int32)
                for j in range(8):
                    xv = plsc.load_gather(xbuf, [rows[j], col])
                    ev = plsc.load_gather(ebuf, [rows[j], col])
                    obuf[i, pl.ds(cs * 128 + j * _L, _L)] = xv + _SCALE * ev

        @pl.loop(0, _NTILE)
        def _write_loop(i):
            s_loc = i // 8
            tc = i % 8
            t = (c // 4) * 8 + (c % 4) * _TT + s_loc
            pltpu.async_copy(obuf.at[i], out_hbm.at[t, tc, wid], sems_w[par])

    @pl.loop(0, _TR)
    def _stage(tr):
        pltpu.sync_copy(idx_hbm.at[tr, wid], idx_v.at[tr])

    fire_gathers(0, 0)

    @pl.loop(0, _NCHUNK, step=2)
    def _chunk_loop(g):
        do_chunk(g, 0)
        do_chunk(g + 1, 1)

    drain_writes(0)
    drain_writes(1)


def _make_sc_kernel():
    mesh = plsc.VectorSubcoreMesh(
        core_axis_name="c", subcore_axis_name="s",
        num_cores=_NC, num_subcores=_NS)
    return pl.kernel(
        _sc_body,
        out_type=jax.ShapeDtypeStruct((_T, 8, _NW, 1024), jnp.float32),
        mesh=mesh,
        compiler_params=pltpu.CompilerParams(
            use_tc_tiling_on_sc=False, needs_layout_passes=False),
        scratch_types=[
            pltpu.VMEM((_TR, 1024), jnp.int32),
            [pltpu.VMEM((_RC, _D), jnp.float32) for _ in range(2)],
            [pltpu.VMEM((_RC, _D), jnp.float32) for _ in range(2)],
            [pltpu.VMEM((_NTILE, 1024), jnp.float32) for _ in range(2)],
            [pltpu.SemaphoreType.DMA for _ in range(2)],
            [pltpu.SemaphoreType.DMA for _ in range(2)],
            [pltpu.SemaphoreType.DMA for _ in range(2)],
        ],
    )


def kernel(tokens, base_table, table):
    t4 = (jnp.asarray(tokens, jnp.int32)
          .reshape(_NW, 128, _TR, 8)
          .transpose(2, 0, 3, 1)
          .reshape(_TR, _NW, 1024))
    out5 = _make_sc_kernel()(t4, base_table, table)
    return (out5.reshape(_T, 8, _NW, 8, 128)
            .transpose(2, 4, 0, 1, 3)
            .reshape(_B, _T, _D))


# speedup vs baseline: 1.3836x; 1.2664x over previous
import math

import jax
import jax.numpy as jnp
from jax import lax
from jax.experimental import pallas as pl
from jax.experimental.pallas import tpu as pltpu
from jax.experimental.pallas import tpu_sc as plsc

_D = 64
_SCALE = math.sqrt(_D)
_NC = 2
_NS = 16
_NW = _NC * _NS
_L = 16
_B = 4096
_T = 200
_TT = 2
_RC = _TT * 128
_TR = _T // 8
_NCHUNK = _T // _TT
_NTILE = _TT * 8


def _sc_body(idx_hbm, base_hbm, tab_hbm, out_hbm,
             idx_v, xbufs, ebufs, obufs, sems_x, sems_e, sems_w):
    wid = lax.axis_index("s") * _NC + lax.axis_index("c")
    lane_iota = lax.iota(jnp.int32, _L)

    def idx_slice(c):
        return idx_v.at[c // 4, pl.ds((c % 4) * _RC, _RC)]

    def fire_gathers(c, par):
        pltpu.async_copy(base_hbm.at[idx_slice(c)], xbufs[par], sems_x[par])
        pltpu.async_copy(tab_hbm.at[idx_slice(c)], ebufs[par], sems_e[par])

    def wait_gathers(c, par):
        pltpu.make_async_copy(base_hbm.at[idx_slice(c)], xbufs[par],
                              sems_x[par]).wait()
        pltpu.make_async_copy(tab_hbm.at[idx_slice(c)], ebufs[par],
                              sems_e[par]).wait()

    def drain_writes(par):
        for i in range(_NTILE):
            pltpu.make_async_copy(obufs[par].at[0], out_hbm.at[0, 0, wid],
                                  sems_w[par]).wait()

    def do_chunk(c, par):
        @pl.when(c + 1 < _NCHUNK)
        def _():
            fire_gathers(c + 1, 1 - par)

        wait_gathers(c, par)

        @pl.when(c >= 2)
        def _():
            drain_writes(par)

        xbuf, ebuf, obuf = xbufs[par], ebufs[par], obufs[par]

        @plsc.parallel_loop(0, _NTILE, unroll=2)
        def _tile_loop(i):
            s_loc = i // 8
            tc = i % 8
            row0 = s_loc * 128
            rows = [row0 + j * _L + lane_iota for j in range(8)]
            for cs in range(8):
                col = jnp.full((_L,), tc * 8 + cs, jnp.int32)
                for j in range(8):
                    xv = plsc.load_gather(xbuf, [rows[j], col])
                    ev = plsc.load_gather(ebuf, [rows[j], col])
                    obuf[i, pl.ds(cs * 128 + j * _L, _L)] = xv + _SCALE * ev

        @pl.loop(0, _NTILE)
        def _write_loop(i):
            s_loc = i // 8
            tc = i % 8
            t = (c // 4) * 8 + (c % 4) * _TT + s_loc
            pltpu.async_copy(obuf.at[i], out_hbm.at[t, tc, wid], sems_w[par])

    @pl.loop(0, _TR)
    def _stage(tr):
        pltpu.sync_copy(idx_hbm.at[tr, wid], idx_v.at[tr])

    fire_gathers(0, 0)

    @pl.loop(0, _NCHUNK, step=2)
    def _chunk_loop(g):
        do_chunk(g, 0)
        do_chunk(g + 1, 1)

    drain_writes(0)
    drain_writes(1)


def _make_sc_kernel():
    mesh = plsc.VectorSubcoreMesh(
        core_axis_name="c", subcore_axis_name="s",
        num_cores=_NC, num_subcores=_NS)
    return pl.kernel(
        _sc_body,
        out_type=jax.ShapeDtypeStruct((_T, 8, _NW, 1024), jnp.float32),
        mesh=mesh,
        compiler_params=pltpu.CompilerParams(
            use_tc_tiling_on_sc=False, needs_layout_passes=False,
            disable_bounds_checks=True, disable_semaphore_checks=True),
        scratch_types=[
            pltpu.VMEM((_TR, 1024), jnp.int32),
            [pltpu.VMEM((_RC, _D), jnp.float32) for _ in range(2)],
            [pltpu.VMEM((_RC, _D), jnp.float32) for _ in range(2)],
            [pltpu.VMEM((_NTILE, 1024), jnp.float32) for _ in range(2)],
            [pltpu.SemaphoreType.DMA for _ in range(2)],
            [pltpu.SemaphoreType.DMA for _ in range(2)],
            [pltpu.SemaphoreType.DMA for _ in range(2)],
        ],
    )


def kernel(tokens, base_table, table):
    t4 = (jnp.asarray(tokens, jnp.int32)
          .reshape(_NW, 128, _TR, 8)
          .transpose(2, 0, 3, 1)
          .reshape(_TR, _NW, 1024))
    out5 = _make_sc_kernel()(t4, base_table, table)
    return (out5.reshape(_T, 8, _NW, 8, 128)
            .transpose(2, 4, 0, 1, 3)
            .reshape(_B, _T, _D))


# speedup vs baseline: 1.9990x; 1.4448x over previous
import math

import jax
import jax.numpy as jnp
from jax import lax
from jax.experimental import pallas as pl
from jax.experimental.pallas import tpu as pltpu
from jax.experimental.pallas import tpu_sc as plsc

_D = 64
_SCALE = math.sqrt(_D)
_NC = 2
_NS = 16
_NW = _NC * _NS
_L = 16
_B = 4096
_T = 200
_TT = 2
_RC = _TT * 128
_TR = _T // 8
_NCHUNK = _T // _TT
_NTILE = _TT * 8


def _sc_body(idx_hbm, base_hbm, tab_hbm, out_hbm,
             idx_bufs, xbufs, ebufs, obufs, sems_i, sems_x, sems_e, sems_w):
    wid = lax.axis_index("s") * _NC + lax.axis_index("c")
    lane_iota = lax.iota(jnp.int32, _L)
    u_pats = [(lane_iota + d) & 15 for d in range(16)]
    w_pats = [(u >> 3) * 1024 + (u & 7) * 128 + lane_iota for u in u_pats]

    def idx_src(c):
        return idx_hbm.at[c // 4, wid, pl.ds((c % 4) * _RC, _RC)]

    def fire_idx(c, par):
        pltpu.async_copy(idx_src(c), idx_bufs[par], sems_i[par])

    def wait_idx(c, par):
        pltpu.make_async_copy(idx_src(c), idx_bufs[par], sems_i[par]).wait()

    def fire_gathers(par):
        pltpu.async_copy(base_hbm.at[idx_bufs[par]], xbufs[par], sems_x[par])
        pltpu.async_copy(tab_hbm.at[idx_bufs[par]], ebufs[par], sems_e[par])

    def wait_gathers(par):
        pltpu.make_async_copy(base_hbm.at[idx_bufs[par]], xbufs[par],
                              sems_x[par]).wait()
        pltpu.make_async_copy(tab_hbm.at[idx_bufs[par]], ebufs[par],
                              sems_e[par]).wait()

    def drain_writes(par):
        for i in range(_NTILE):
            pltpu.make_async_copy(obufs[par].at[pl.ds(0, 1024)],
                                  out_hbm.at[0, 0, wid], sems_w[par]).wait()

    def do_chunk(c, par):
        @pl.when(c + 1 < _NCHUNK)
        def _():
            wait_idx(c + 1, 1 - par)
            fire_gathers(1 - par)

        wait_gathers(par)

        @pl.when(c + 2 < _NCHUNK)
        def _():
            fire_idx(c + 2, par)

        @pl.when(c >= 2)
        def _():
            drain_writes(par)

        xbuf, ebuf, obuf = xbufs[par], ebufs[par], obufs[par]

        @plsc.parallel_loop(0, _L, unroll=2)
        def _tile_loop(m):
            s_loc = m // 8
            j = m % 8
            rows = lane_iota + (s_loc * 128 + j * _L)
            for ci in range(4):
                c0 = ci * _L
                base_w = (s_loc * 8 + c0 // 8) * 1024 + j * _L
                for d in range(16):
                    cols = u_pats[d] + c0
                    widx = w_pats[d] + base_w
                    xv = plsc.load_gather(xbuf, [rows, cols])
                    ev = plsc.load_gather(ebuf, [rows, cols])
                    plsc.store_scatter(obuf, [widx], xv + _SCALE * ev)

        @pl.loop(0, _NTILE)
        def _write_loop(i):
            s_loc = i // 8
            tc = i % 8
            t = (c // 4) * 8 + (c % 4) * _TT + s_loc
            pltpu.async_copy(obuf.at[pl.ds(i * 1024, 1024)],
                             out_hbm.at[t, tc, wid], sems_w[par])

    pltpu.sync_copy(idx_src(0), idx_bufs[0])
    fire_gathers(0)
    fire_idx(1, 1)

    @pl.loop(0, _NCHUNK, step=2)
    def _chunk_loop(g):
        do_chunk(g, 0)
        do_chunk(g + 1, 1)

    drain_writes(0)
    drain_writes(1)


def _make_sc_kernel():
    mesh = plsc.VectorSubcoreMesh(
        core_axis_name="c", subcore_axis_name="s",
        num_cores=_NC, num_subcores=_NS)
    return pl.kernel(
        _sc_body,
        out_type=jax.ShapeDtypeStruct((_T, 8, _NW, 1024), jnp.float32),
        mesh=mesh,
        compiler_params=pltpu.CompilerParams(
            use_tc_tiling_on_sc=False, needs_layout_passes=False,
            disable_bounds_checks=True, disable_semaphore_checks=True),
        scratch_types=[
            [pltpu.VMEM((_RC,), jnp.int32) for _ in range(2)],
            [pltpu.VMEM((_RC, _D), jnp.float32) for _ in range(2)],
            [pltpu.VMEM((_RC, _D), jnp.float32) for _ in range(2)],
            [pltpu.VMEM((_NTILE * 1024,), jnp.float32) for _ in range(2)],
            [pltpu.SemaphoreType.DMA for _ in range(2)],
            [pltpu.SemaphoreType.DMA for _ in range(2)],
            [pltpu.SemaphoreType.DMA for _ in range(2)],
            [pltpu.SemaphoreType.DMA for _ in range(2)],
        ],
    )


def kernel(tokens, base_table, table):
    t4 = (jnp.asarray(tokens, jnp.int32)
          .reshape(_NW, 128, _TR, 8)
          .transpose(2, 0, 3, 1)
          .reshape(_TR, _NW, 1024))
    out5 = _make_sc_kernel()(t4, base_table, table)
    return (out5.reshape(_T, 8, _NW, 8, 128)
            .transpose(2, 4, 0, 1, 3)
            .reshape(_B, _T, _D))
